# baseline (speedup 1.0000x reference)
import jax
import jax.numpy as jnp
from jax import lax
from jax.experimental import pallas as pl
from jax.experimental.pallas import tpu as pltpu
from jax.experimental.pallas import tpu_sc as plsc

N = 32768
B = 16
D = 128
GC = 512
NEG = -1e30

NCORE = 2
NSUB = 16
NW = NCORE * NSUB
TPW = N // NW
CHT = 128
NCH = TPW // CHT
GRP = CHT // 16


def _sc_body(x_hbm, qt_hbm, w_hbm, out_hbm,
             w_v, xb0, xb1, qb0, qb1, lb, sx0, sx1, sq0, sq1):
    wid = lax.axis_index("s") * NCORE + lax.axis_index("c")
    tok0 = wid * TPW

    pltpu.sync_copy(w_hbm, w_v)

    xbufs = (xb0, xb1)
    qbufs = (qb0, qb1)
    sxs = (sx0, sx1)
    sqs = (sq0, sq1)

    def issue(c, b):
        pltpu.async_copy(x_hbm.at[pl.ds((tok0 + c * CHT) * D, CHT * D)],
                         xbufs[b], sxs[b])
        pltpu.async_copy(qt_hbm.at[:, pl.ds(tok0 + c * CHT, CHT)],
                         qbufs[b], sqs[b])

    issue(0, 0)
    issue(1, 1)

    lane = jnp.arange(16, dtype=jnp.int32)
    dconst = [db * 16 + lane for db in range(D // 32)]
    perm = [lane ^ s for s in (8, 4, 2, 1)]

    def hsum(v):
        for p in perm:
            v = v + v.at[p].get(mode="promise_in_bounds")
        return v

    def outer(g, carry):
        for b in range(2):
            c = g * 2 + b
            pltpu.make_async_copy(x_hbm.at[pl.ds(0, CHT * D)],
                                  xbufs[b], sxs[b]).wait()
            pltpu.make_async_copy(qt_hbm.at[:, pl.ds(0, CHT)],
                                  qbufs[b], sqs[b]).wait()

            def grp_body(t, carry2, b=b, c=c):
                qvs = [qbufs[b][l, pl.ds(t * 16, 16)] * (D // 2)
                       for l in range(3)]
                lane0 = lane == 0
                for j in range(16):
                    qoff = [qvs[l].at[jnp.full((16,), j, jnp.int32)]
                            .get(mode="promise_in_bounds")
                            for l in range(3)]
                    accs = [[jnp.zeros((16,), jnp.float32) for _ in range(3)]
                            for _ in range(4)]
                    for db in range(D // 32):
                        bb = t * (16 * D) + j * D + db * 32
                        xa = xbufs[b][pl.ds(bb, 16)]
                        xb = xbufs[b][pl.ds(bb + 16, 16)]
                        for l in range(3):
                            wg = plsc.load_gather(w_v, [qoff[l] + dconst[db]])
                            wa, wb = plsc.unpack(
                                plsc.bitcast(wg, jnp.bfloat16),
                                format=plsc.PackFormat.INTERLEAVED)
                            accs[db % 2][l] = accs[db % 2][l] + xa * wa
                            accs[2 + db % 2][l] = accs[2 + db % 2][l] + xb * wb
                    for l in range(3):
                        tot = (accs[0][l] + accs[1][l]) + (accs[2][l] + accs[3][l])
                        pos = l * TPW + c * CHT + t * 16 + j
                        plsc.store_scatter(lb, [jnp.full((16,), pos, jnp.int32)],
                                           hsum(tot), mask=lane0)
                return carry2

            lax.fori_loop(0, GRP, grp_body, 0)

            @pl.when(c + 2 < NCH)
            def _prefetch(b=b, c=c):
                issue(c + 2, b)
        return carry

    lax.fori_loop(0, NCH // 2, outer, 0)
    for l in range(3):
        pltpu.sync_copy(lb.at[pl.ds(l * TPW, TPW)],
                        out_hbm.at[pl.ds(l * N + tok0, TPW)])


def _sc_logits(x_flat, q_t, w_pack):
    return pl.kernel(
        _sc_body,
        out_type=jax.ShapeDtypeStruct((3 * N,), jnp.float32),
        mesh=plsc.VectorSubcoreMesh(core_axis_name="c", subcore_axis_name="s"),
        compiler_params=pltpu.CompilerParams(needs_layout_passes=False),
        scratch_types=[
            pltpu.VMEM((GC * D // 2,), jnp.int32),
            pltpu.VMEM((CHT * D,), jnp.float32),
            pltpu.VMEM((CHT * D,), jnp.float32),
            pltpu.VMEM((3, CHT), jnp.int32),
            pltpu.VMEM((3, CHT), jnp.int32),
            pltpu.VMEM((3 * TPW,), jnp.float32),
            pltpu.SemaphoreType.DMA,
            pltpu.SemaphoreType.DMA,
            pltpu.SemaphoreType.DMA,
            pltpu.SemaphoreType.DMA,
        ],
    )(x_flat, q_t, w_pack)


CH = 8192
NCHUNK = N // CH


def _tc_body(lg_ref, x_ref, cu_ref, out_ref, m_ref, d_ref, a_ref):
    i = pl.program_id(0)

    @pl.when(i == 0)
    def _init():
        m_ref[...] = jnp.full((3, B), NEG, jnp.float32)
        d_ref[...] = jnp.zeros((3, B), jnp.float32)
        a_ref[...] = jnp.zeros((3, B, D), jnp.float32)

    x_c = x_ref[...]
    tok = i * CH + lax.broadcasted_iota(jnp.int32, (1, CH), 1)
    cu = jnp.stack([cu_ref[bb] for bb in range(B + 1)])
    lo = cu[:B][:, None]
    hi = cu[1:][:, None]
    mask = (tok >= lo) & (tok < hi)
    maskf = mask.astype(jnp.float32)
    lg = lg_ref[...]

    for l in range(3):
        logit = lg[l:l + 1, :]
        lmask = jnp.where(mask, logit, NEG)
        cmax = jnp.max(lmask, axis=1)
        m_old = m_ref[l, :]
        m_new = jnp.maximum(m_old, cmax)
        scale = jnp.exp(m_old - m_new)
        m_tok = lax.dot_general(m_new, maskf, (((0,), (0,)), ((), ())),
                                preferred_element_type=jnp.float32)
        e_row = jnp.exp(logit - m_tok[None, :])
        ew = maskf * e_row
        d_new = d_ref[l, :] * scale + jnp.sum(ew, axis=1)
        contrib = jnp.dot(ew, x_c, preferred_element_type=jnp.float32)
        a_new = a_ref[l] * scale[:, None] + contrib
        m_ref[l, :] = m_new
        d_ref[l, :] = d_new
        a_ref[l] = a_new

        @pl.when(i == NCHUNK - 1)
        def _fin():
            denom = d_new[:, None]
            out_ref[l] = jnp.where(denom > 0.0, a_new / denom, 0.0)


def _tc_pool(logits, x, cu_seqlens):
    return pl.pallas_call(
        _tc_body,
        grid=(NCHUNK,),
        in_specs=[
            pl.BlockSpec((3, CH), lambda i: (0, i)),
            pl.BlockSpec((CH, D), lambda i: (i, 0)),
            pl.BlockSpec(memory_space=pltpu.SMEM),
        ],
        out_specs=pl.BlockSpec((3, B, D), lambda i: (0, 0, 0)),
        out_shape=jax.ShapeDtypeStruct((3, B, D), jnp.float32),
        scratch_shapes=[
            pltpu.VMEM((3, B), jnp.float32),
            pltpu.VMEM((3, B), jnp.float32),
            pltpu.VMEM((3, B, D), jnp.float32),
        ],
    )(logits, x, cu_seqlens)


@jax.jit
def _run(x, attention_query, cu_seqlens, attn_weight):
    q_t = attention_query.T
    w4 = attn_weight.astype(jnp.bfloat16).reshape(GC, D // 32, 2, 16)
    pairs = jnp.stack([w4[:, :, 0, :], w4[:, :, 1, :]], axis=-1)
    w_pack = lax.bitcast_convert_type(pairs, jnp.int32).reshape(GC * D // 2)
    logits = _sc_logits(x.reshape(N * D), q_t, w_pack).reshape(3, N)
    return _tc_pool(logits, x, cu_seqlens)


def kernel(x, attention_query, cu_seqlens, attn_weight):
    return (_run(x, attention_query, cu_seqlens, attn_weight), None, None)


# speedup vs baseline: 1.2041x; 1.2041x over previous
import jax
import jax.numpy as jnp
from jax import lax
from jax.experimental import pallas as pl
from jax.experimental.pallas import tpu as pltpu
from jax.experimental.pallas import tpu_sc as plsc

N = 32768
B = 16
D = 128
GC = 512
NEG = -1e30

NCORE = 2
NSUB = 16
NW = NCORE * NSUB
TPW = N // NW
CHT = 128
NCH = TPW // CHT
GRP = CHT // 16


def _sc_body(x_hbm, qt_hbm, w_hbm, out_hbm,
             w_v, xb0, xb1, qb0, qb1, lb, sx0, sx1, sq0, sq1):
    wid = lax.axis_index("s") * NCORE + lax.axis_index("c")
    tok0 = wid * TPW

    pltpu.sync_copy(w_hbm, w_v)

    xbufs = (xb0, xb1)
    qbufs = (qb0, qb1)
    sxs = (sx0, sx1)
    sqs = (sq0, sq1)

    def issue(c, b):
        pltpu.async_copy(x_hbm.at[pl.ds((tok0 + c * CHT) * D, CHT * D)],
                         xbufs[b], sxs[b])
        pltpu.async_copy(qt_hbm.at[:, pl.ds(tok0 + c * CHT, CHT)],
                         qbufs[b], sqs[b])

    issue(0, 0)
    issue(1, 1)

    lane = jnp.arange(16, dtype=jnp.int32)
    dconst = [db * 16 + lane for db in range(D // 32)]
    perm = [lane ^ s for s in (8, 4, 2, 1)]

    def hsum(v):
        for p in perm:
            v = v + v.at[p].get(mode="promise_in_bounds")
        return v

    def outer(g, carry):
        for b in range(2):
            c = g * 2 + b
            pltpu.make_async_copy(x_hbm.at[pl.ds(0, CHT * D)],
                                  xbufs[b], sxs[b]).wait()
            pltpu.make_async_copy(qt_hbm.at[:, pl.ds(0, CHT)],
                                  qbufs[b], sqs[b]).wait()

            def grp_body(t, carry2, b=b, c=c):
                qvs = [qbufs[b][l, pl.ds(t * 16, 16)] * (D // 2)
                       for l in range(3)]
                res = [jnp.zeros((16,), jnp.float32) for _ in range(3)]
                for j in range(16):
                    qoff = [qvs[l].at[jnp.full((16,), j, jnp.int32)]
                            .get(mode="promise_in_bounds")
                            for l in range(3)]
                    accs = [[jnp.zeros((16,), jnp.float32) for _ in range(3)]
                            for _ in range(4)]
                    for db in range(D // 32):
                        bb = t * (16 * D) + j * D + db * 32
                        xa = xbufs[b][pl.ds(bb, 16)]
                        xb = xbufs[b][pl.ds(bb + 16, 16)]
                        for l in range(3):
                            wg = plsc.load_gather(w_v, [qoff[l] + dconst[db]])
                            wa, wb = plsc.unpack(
                                plsc.bitcast(wg, jnp.bfloat16),
                                format=plsc.PackFormat.INTERLEAVED)
                            accs[db % 2][l] = accs[db % 2][l] + xa * wa
                            accs[2 + db % 2][l] = accs[2 + db % 2][l] + xb * wb
                    sel = lane == j
                    for l in range(3):
                        tot = (accs[0][l] + accs[1][l]) + (accs[2][l] + accs[3][l])
                        res[l] = jnp.where(sel, hsum(tot), res[l])
                for l in range(3):
                    lb[pl.ds(l * TPW + c * CHT + t * 16, 16)] = res[l]
                return carry2

            lax.fori_loop(0, GRP, grp_body, 0)

            @pl.when(c + 2 < NCH)
            def _prefetch(b=b, c=c):
                issue(c + 2, b)
        return carry

    lax.fori_loop(0, NCH // 2, outer, 0)
    for l in range(3):
        pltpu.sync_copy(lb.at[pl.ds(l * TPW, TPW)],
                        out_hbm.at[pl.ds(l * N + tok0, TPW)])


def _sc_logits(x_flat, q_t, w_pack):
    return pl.kernel(
        _sc_body,
        out_type=jax.ShapeDtypeStruct((3 * N,), jnp.float32),
        mesh=plsc.VectorSubcoreMesh(core_axis_name="c", subcore_axis_name="s"),
        compiler_params=pltpu.CompilerParams(needs_layout_passes=False),
        scratch_types=[
            pltpu.VMEM((GC * D // 2,), jnp.int32),
            pltpu.VMEM((CHT * D,), jnp.float32),
            pltpu.VMEM((CHT * D,), jnp.float32),
            pltpu.VMEM((3, CHT), jnp.int32),
            pltpu.VMEM((3, CHT), jnp.int32),
            pltpu.VMEM((3 * TPW,), jnp.float32),
            pltpu.SemaphoreType.DMA,
            pltpu.SemaphoreType.DMA,
            pltpu.SemaphoreType.DMA,
            pltpu.SemaphoreType.DMA,
        ],
    )(x_flat, q_t, w_pack)


CH = 8192
NCHUNK = N // CH


def _tc_body(lg_ref, x_ref, cu_ref, out_ref, m_ref, d_ref, a_ref):
    i = pl.program_id(0)

    @pl.when(i == 0)
    def _init():
        m_ref[...] = jnp.full((3, B), NEG, jnp.float32)
        d_ref[...] = jnp.zeros((3, B), jnp.float32)
        a_ref[...] = jnp.zeros((3, B, D), jnp.float32)

    x_c = x_ref[...]
    tok = i * CH + lax.broadcasted_iota(jnp.int32, (1, CH), 1)
    cu = jnp.stack([cu_ref[bb] for bb in range(B + 1)])
    lo = cu[:B][:, None]
    hi = cu[1:][:, None]
    mask = (tok >= lo) & (tok < hi)
    maskf = mask.astype(jnp.float32)
    lg = lg_ref[...]

    for l in range(3):
        logit = lg[l:l + 1, :]
        lmask = jnp.where(mask, logit, NEG)
        cmax = jnp.max(lmask, axis=1)
        m_old = m_ref[l, :]
        m_new = jnp.maximum(m_old, cmax)
        scale = jnp.exp(m_old - m_new)
        m_tok = lax.dot_general(m_new, maskf, (((0,), (0,)), ((), ())),
                                preferred_element_type=jnp.float32)
        e_row = jnp.exp(logit - m_tok[None, :])
        ew = maskf * e_row
        d_new = d_ref[l, :] * scale + jnp.sum(ew, axis=1)
        contrib = jnp.dot(ew, x_c, preferred_element_type=jnp.float32)
        a_new = a_ref[l] * scale[:, None] + contrib
        m_ref[l, :] = m_new
        d_ref[l, :] = d_new
        a_ref[l] = a_new

        @pl.when(i == NCHUNK - 1)
        def _fin():
            denom = d_new[:, None]
            out_ref[l] = jnp.where(denom > 0.0, a_new / denom, 0.0)


def _tc_pool(logits, x, cu_seqlens):
    return pl.pallas_call(
        _tc_body,
        grid=(NCHUNK,),
        in_specs=[
            pl.BlockSpec((3, CH), lambda i: (0, i)),
            pl.BlockSpec((CH, D), lambda i: (i, 0)),
            pl.BlockSpec(memory_space=pltpu.SMEM),
        ],
        out_specs=pl.BlockSpec((3, B, D), lambda i: (0, 0, 0)),
        out_shape=jax.ShapeDtypeStruct((3, B, D), jnp.float32),
        scratch_shapes=[
            pltpu.VMEM((3, B), jnp.float32),
            pltpu.VMEM((3, B), jnp.float32),
            pltpu.VMEM((3, B, D), jnp.float32),
        ],
    )(logits, x, cu_seqlens)


@jax.jit
def _run(x, attention_query, cu_seqlens, attn_weight):
    q_t = attention_query.T
    w4 = attn_weight.astype(jnp.bfloat16).reshape(GC, D // 32, 2, 16)
    pairs = jnp.stack([w4[:, :, 0, :], w4[:, :, 1, :]], axis=-1)
    w_pack = lax.bitcast_convert_type(pairs, jnp.int32).reshape(GC * D // 2)
    logits = _sc_logits(x.reshape(N * D), q_t, w_pack).reshape(3, N)
    return _tc_pool(logits, x, cu_seqlens)


def kernel(x, attention_query, cu_seqlens, attn_weight):
    return (_run(x, attention_query, cu_seqlens, attn_weight), None, None)


# speedup vs baseline: 1.2258x; 1.0180x over previous
import jax
import jax.numpy as jnp
from jax import lax
from jax.experimental import pallas as pl
from jax.experimental.pallas import tpu as pltpu
from jax.experimental.pallas import tpu_sc as plsc

N = 32768
B = 16
D = 128
GC = 512
NEG = -1e30

NCORE = 2
NSUB = 16
NW = NCORE * NSUB
TPW = N // NW
CHT = 128
NCH = TPW // CHT
GRP = CHT // 16


def _sc_body(x_hbm, qt_hbm, w_hbm, out_hbm,
             w_v, xb0, xb1, qb0, qb1, lb, sx0, sx1, sq0, sq1):
    wid = lax.axis_index("s") * NCORE + lax.axis_index("c")
    tok0 = wid * TPW

    pltpu.sync_copy(w_hbm, w_v)

    xbufs = (xb0, xb1)
    qbufs = (qb0, qb1)
    sxs = (sx0, sx1)
    sqs = (sq0, sq1)

    def issue(c, b):
        pltpu.async_copy(x_hbm.at[pl.ds((tok0 + c * CHT) * D, CHT * D)],
                         xbufs[b], sxs[b])
        pltpu.async_copy(qt_hbm.at[:, pl.ds(tok0 + c * CHT, CHT)],
                         qbufs[b], sqs[b])

    issue(0, 0)
    issue(1, 1)

    lane = jnp.arange(16, dtype=jnp.int32)
    dconst = [db * 16 + lane for db in range(D // 32)]
    perm = [lane ^ s for s in (8, 4, 2, 1)]

    def hsum(v):
        for p in perm:
            v = v + v.at[p].get(mode="promise_in_bounds")
        return v

    def outer(g, carry):
        for b in range(2):
            c = g * 2 + b
            pltpu.make_async_copy(x_hbm.at[pl.ds(0, CHT * D)],
                                  xbufs[b], sxs[b]).wait()
            pltpu.make_async_copy(qt_hbm.at[:, pl.ds(0, CHT)],
                                  qbufs[b], sqs[b]).wait()

            def grp_body(t, carry2, b=b, c=c):
                qvs = [qbufs[b][l, pl.ds(t * 16, 16)] for l in range(3)]
                res = [jnp.zeros((16,), jnp.float32) for _ in range(3)]
                for j in range(16):
                    qoff = [qvs[l].at[jnp.full((16,), j, jnp.int32)]
                            .get(mode="promise_in_bounds") * (D // 2)
                            for l in range(3)]
                    accs = [[jnp.zeros((16,), jnp.float32) for _ in range(3)]
                            for _ in range(4)]
                    for db in range(D // 32):
                        bb = t * (16 * D) + j * D + db * 32
                        xa = xbufs[b][pl.ds(bb, 16)]
                        xb = xbufs[b][pl.ds(bb + 16, 16)]
                        for l in range(3):
                            wg = plsc.load_gather(w_v, [qoff[l] + dconst[db]])
                            wa, wb = plsc.unpack(
                                plsc.bitcast(wg, jnp.bfloat16),
                                format=plsc.PackFormat.INTERLEAVED)
                            accs[db % 2][l] = accs[db % 2][l] + xa * wa
                            accs[2 + db % 2][l] = accs[2 + db % 2][l] + xb * wb
                    sel = lane == j
                    for l in range(3):
                        tot = (accs[0][l] + accs[1][l]) + (accs[2][l] + accs[3][l])
                        res[l] = jnp.where(sel, hsum(tot), res[l])
                for l in range(3):
                    lb[l, pl.ds(c * CHT + t * 16, 16)] = res[l]
                return carry2

            lax.fori_loop(0, GRP, grp_body, 0)

            @pl.when(c + 2 < NCH)
            def _prefetch(b=b, c=c):
                issue(c + 2, b)
        return carry

    lax.fori_loop(0, NCH // 2, outer, 0)
    pltpu.sync_copy(lb, out_hbm.at[:, pl.ds(tok0, TPW)])


def _sc_logits(x_flat, q_t, w_pack):
    return pl.kernel(
        _sc_body,
        out_type=jax.ShapeDtypeStruct((3, N), jnp.float32),
        mesh=plsc.VectorSubcoreMesh(core_axis_name="c", subcore_axis_name="s"),
        compiler_params=pltpu.CompilerParams(needs_layout_passes=False),
        scratch_types=[
            pltpu.VMEM((GC * D // 2,), jnp.int32),
            pltpu.VMEM((CHT * D,), jnp.float32),
            pltpu.VMEM((CHT * D,), jnp.float32),
            pltpu.VMEM((3, CHT), jnp.int32),
            pltpu.VMEM((3, CHT), jnp.int32),
            pltpu.VMEM((3, TPW), jnp.float32),
            pltpu.SemaphoreType.DMA,
            pltpu.SemaphoreType.DMA,
            pltpu.SemaphoreType.DMA,
            pltpu.SemaphoreType.DMA,
        ],
    )(x_flat, q_t, w_pack)


CH = 8192
NCHUNK = N // CH


def _tc_body(lg_ref, x_ref, cu_ref, out_ref, m_ref, d_ref, a_ref):
    i = pl.program_id(0)

    @pl.when(i == 0)
    def _init():
        m_ref[...] = jnp.full((3, B), NEG, jnp.float32)
        d_ref[...] = jnp.zeros((3, B), jnp.float32)
        a_ref[...] = jnp.zeros((3, B, D), jnp.float32)

    x_c = x_ref[...]
    tok = i * CH + lax.broadcasted_iota(jnp.int32, (1, CH), 1)
    cu = jnp.stack([cu_ref[bb] for bb in range(B + 1)])
    lo = cu[:B][:, None]
    hi = cu[1:][:, None]
    mask = (tok >= lo) & (tok < hi)
    maskf = mask.astype(jnp.float32)
    lg = lg_ref[...]

    for l in range(3):
        logit = lg[l:l + 1, :]
        lmask = jnp.where(mask, logit, NEG)
        cmax = jnp.max(lmask, axis=1)
        m_old = m_ref[l, :]
        m_new = jnp.maximum(m_old, cmax)
        scale = jnp.exp(m_old - m_new)
        m_tok = lax.dot_general(m_new, maskf, (((0,), (0,)), ((), ())),
                                preferred_element_type=jnp.float32)
        e_row = jnp.exp(logit - m_tok[None, :])
        ew = maskf * e_row
        d_new = d_ref[l, :] * scale + jnp.sum(ew, axis=1)
        contrib = jnp.dot(ew, x_c, preferred_element_type=jnp.float32)
        a_new = a_ref[l] * scale[:, None] + contrib
        m_ref[l, :] = m_new
        d_ref[l, :] = d_new
        a_ref[l] = a_new

        @pl.when(i == NCHUNK - 1)
        def _fin():
            denom = d_new[:, None]
            out_ref[l] = jnp.where(denom > 0.0, a_new / denom, 0.0)


def _tc_pool(logits, x, cu_seqlens):
    return pl.pallas_call(
        _tc_body,
        grid=(NCHUNK,),
        in_specs=[
            pl.BlockSpec((3, CH), lambda i: (0, i)),
            pl.BlockSpec((CH, D), lambda i: (i, 0)),
            pl.BlockSpec(memory_space=pltpu.SMEM),
        ],
        out_specs=pl.BlockSpec((3, B, D), lambda i: (0, 0, 0)),
        out_shape=jax.ShapeDtypeStruct((3, B, D), jnp.float32),
        scratch_shapes=[
            pltpu.VMEM((3, B), jnp.float32),
            pltpu.VMEM((3, B), jnp.float32),
            pltpu.VMEM((3, B, D), jnp.float32),
        ],
    )(logits, x, cu_seqlens)


@jax.jit
def _run(x, attention_query, cu_seqlens, attn_weight):
    q_t = attention_query.T
    w4 = attn_weight.astype(jnp.bfloat16).reshape(GC, D // 32, 2, 16)
    pairs = jnp.stack([w4[:, :, 0, :], w4[:, :, 1, :]], axis=-1)
    w_pack = lax.bitcast_convert_type(pairs, jnp.int32).reshape(GC * D // 2)
    logits = _sc_logits(x.reshape(N * D), q_t, w_pack)
    return _tc_pool(logits, x, cu_seqlens)


def kernel(x, attention_query, cu_seqlens, attn_weight):
    return (_run(x, attention_query, cu_seqlens, attn_weight), None, None)
